# baseline (speedup 1.0000x reference)
import functools

import jax
import jax.numpy as jnp
from jax import lax
from jax.experimental import pallas as pl
from jax.experimental.pallas import tpu as pltpu
from jax.experimental.pallas import tpu_sc as plsc

H, W = 180, 240
S = 32
START_IDX = 2
B = 4
N = 1048576
SEG = N // S
HW = H * W
HV, WV = H // 2, W // 2

HP, WP = 184, 256
HVP, WVP = HP // 2, WP // 2
NC, NS, LANES = 2, 16, 16
NW = NC * NS
PAIRS = B * S
PPW = PAIRS // 2 // NW
CHUNK = 8192
NCHUNK = SEG // CHUNK


CH = 16384
NCH = SEG // CH
NBLK = CH // 128
PLANE = 4 * 8192


def _scatter_body(g, ev_hbm, craw_hbm, bx0, bx1, by0, by1, hist,
                  sx0, sx1, sy0, sy1):
    wid = lax.axis_index("s") * NC + lax.axis_index("c")
    zeros16 = jnp.zeros((LANES,), jnp.int32)
    ones16 = jnp.ones((LANES,), jnp.int32)
    bxs, bys = (bx0, bx1), (by0, by1)
    sxs, syss = (sx0, sx1), (sy0, sy1)
    p0 = wid * PPW
    nslot = PPW * NCH

    def start(t):
        p = g * (PAIRS // 2) + p0 + t // NCH
        b = p // S
        k0 = (p % S) * (SEG // 128) + (t % NCH) * NBLK
        slot = t % 2
        return (pltpu.async_copy(ev_hbm.at[0, pl.ds(k0, NBLK), b],
                                 bxs[slot], sxs[slot]),
                pltpu.async_copy(ev_hbm.at[1, pl.ds(k0, NBLK), b],
                                 bys[slot], syss[slot]))

    @plsc.parallel_loop(0, HP, unroll=2)
    def zero0(r):
        for c in range(WP // LANES):
            hist[r, pl.ds(c * LANES, LANES)] = zeros16

    pend = [start(0)]
    for t in range(nslot):
        if t + 1 < nslot:
            pend.append(start(t + 1))
        cx, cy = pend[t]
        cx.wait()
        cy.wait()
        bx, by = bxs[t % 2], bys[t % 2]

        @plsc.parallel_loop(0, NBLK, unroll=4)
        def scatter_blk(r):
            for c8 in range(8):
                xv = bx[r, pl.ds(c8 * LANES, LANES)].astype(jnp.int32)
                yv = by[r, pl.ds(c8 * LANES, LANES)].astype(jnp.int32)
                plsc.addupdate_scatter(hist, [yv, xv], ones16)

        if t % NCH == NCH - 1:
            pltpu.sync_copy(
                hist, craw_hbm.at[pl.ds((p0 + t // NCH) * HP, HP)])


def _make_scatter(g):
    return functools.partial(
        pl.kernel,
        out_type=jax.ShapeDtypeStruct((PAIRS // 2 * HP, WP), jnp.int32),
        mesh=plsc.VectorSubcoreMesh(core_axis_name="c", subcore_axis_name="s"),
        compiler_params=pltpu.CompilerParams(needs_layout_passes=False),
        scratch_types=[
            pltpu.VMEM((NBLK, 128), jnp.float32),
            pltpu.VMEM((NBLK, 128), jnp.float32),
            pltpu.VMEM((NBLK, 128), jnp.float32),
            pltpu.VMEM((NBLK, 128), jnp.float32),
            pltpu.VMEM((HP, WP), jnp.int32),
            pltpu.SemaphoreType.DMA,
            pltpu.SemaphoreType.DMA,
            pltpu.SemaphoreType.DMA,
            pltpu.SemaphoreType.DMA,
        ],
    )(functools.partial(_scatter_body, g))


_scatter0 = _make_scatter(0)
_scatter1 = _make_scatter(1)


def _aligned_calc(a, D):
    n = S * D
    mean = jnp.sum(a) / n
    var = jnp.sum((a - mean) ** 2) / (n - 1)
    clamp_val = mean + 3.0 * jnp.sqrt(var)
    a = jnp.clip(a, 0.0, clamp_val)
    iS = lax.broadcasted_iota(jnp.int32, (S, S), 0)
    jS = lax.broadcasted_iota(jnp.int32, (S, S), 1)
    TS = (jnp.abs(iS - jS) <= 1).astype(jnp.float32)
    iD = lax.broadcasted_iota(jnp.int32, (D, D), 0)
    jD = lax.broadcasted_iota(jnp.int32, (D, D), 1)
    TD = (jnp.abs(iD - jD) <= 1).astype(jnp.float32)
    box = jnp.dot(jnp.dot(TS, a, preferred_element_type=jnp.float32), TD,
                  preferred_element_type=jnp.float32)
    blur = 0.0625 * box + (0.5 - 0.0625) * a
    dcol = lax.broadcasted_iota(jnp.int32, (D, 1), 0).astype(jnp.float32)
    m = jnp.dot(blur, dcol, preferred_element_type=jnp.float32) / float(SEG)
    sel = lax.broadcasted_iota(jnp.int32, (S, 1), 0) == START_IDX
    start = jnp.sum(jnp.where(sel, m, 0.0))
    dist = (D // 2) - start
    return jnp.round(m - start - dist)


def _post_body(craw_ref, out_ref):
    c_all = craw_ref[0]
    ax_cum = jnp.sum(c_all, axis=1).astype(jnp.float32)[:, :W]
    ay_cum = jnp.sum(c_all, axis=2).astype(jnp.float32)[:, :H]
    seg_iota_x = lax.broadcasted_iota(jnp.int32, (S, W), 0)
    seg_iota_y = lax.broadcasted_iota(jnp.int32, (S, H), 0)
    ax_prev = jnp.concatenate([jnp.zeros((1, W), jnp.float32), ax_cum[:-1]], 0)
    ay_prev = jnp.concatenate([jnp.zeros((1, H), jnp.float32), ay_cum[:-1]], 0)
    along_x = jnp.where(seg_iota_x % PPW == 0, ax_cum, ax_cum - ax_prev)
    along_y = jnp.where(seg_iota_y % PPW == 0, ay_cum, ay_cum - ay_prev)
    a_x = _aligned_calc(along_x, W)
    a_y = _aligned_calc(along_y, H)

    xi = lax.broadcasted_iota(jnp.int32, (WP, WP), 0).astype(jnp.float32)
    xo = lax.broadcasted_iota(jnp.int32, (WP, WP), 1).astype(jnp.float32)
    yi = lax.broadcasted_iota(jnp.int32, (HP, HP), 1).astype(jnp.float32)
    yo = lax.broadcasted_iota(jnp.int32, (HP, HP), 0).astype(jnp.float32)
    qy_i = lax.broadcasted_iota(jnp.int32, (HVP, HP), 1)
    qy_o = lax.broadcasted_iota(jnp.int32, (HVP, HP), 0)
    Qy = (qy_i // 2 == qy_o).astype(jnp.float32)
    qx_i = lax.broadcasted_iota(jnp.int32, (WP, WVP), 0)
    qx_o = lax.broadcasted_iota(jnp.int32, (WP, WVP), 1)
    Qx = (qx_i // 2 == qx_o).astype(jnp.float32)
    sel_iota = lax.broadcasted_iota(jnp.int32, (S, 1), 0)

    def shifted(si):
        sel = sel_iota == si
        ax = jnp.sum(jnp.where(sel, a_x, 0.0))
        ay = jnp.sum(jnp.where(sel, a_y, 0.0))
        Mx = (jnp.clip(xi - ax, 0.0, W - 1.0) == xo).astype(jnp.float32)
        MyT = (jnp.clip(yi - ay, 0.0, H - 1.0) == yo).astype(jnp.float32)
        c_cur = craw_ref[0, si]
        c_prev = craw_ref[0, jnp.maximum(si - 1, 0)]
        keep_prev = jnp.where(jnp.int32(si) % PPW == 0, jnp.int32(0), jnp.int32(1))
        cs = (c_cur - keep_prev * c_prev).astype(jnp.float32)
        sh = jnp.dot(MyT, jnp.dot(cs, Mx, preferred_element_type=jnp.float32),
                     preferred_element_type=jnp.float32)
        occ = (jnp.dot(jnp.dot(Qy, sh, preferred_element_type=jnp.float32), Qx,
                       preferred_element_type=jnp.float32) > 0.0).astype(jnp.float32)
        return sh, occ

    cont0, v0 = shifted(START_IDX)

    def cond(carry):
        si, active, _, _ = carry
        return jnp.logical_and(active, si < S)

    def body(carry):
        si, _, v, cont = carry
        sh, occ = shifted(si)
        vn = jnp.maximum(v, occ)
        vn_cnt = jnp.sum(vn)
        new_info = vn_cnt - jnp.sum(v)
        active = (new_info / vn_cnt) >= 0.01
        cont = jnp.where(active, cont + sh, cont)
        v = jnp.where(active, vn, v)
        return si + 1, active, v, cont

    _, _, _, cont = lax.while_loop(
        cond, body, (jnp.int32(START_IDX + 1), jnp.bool_(True), v0, cont0))
    out_ref[0, 0] = cont[:H, :W]


_post = pl.pallas_call(
    _post_body,
    grid=(B // 2,),
    in_specs=[pl.BlockSpec((1, S, HP, WP), lambda i: (i, 0, 0, 0))],
    out_specs=pl.BlockSpec((1, 1, H, W), lambda i: (i, 0, 0, 0)),
    out_shape=jax.ShapeDtypeStruct((B // 2, 1, H, W), jnp.float32),
)


def kernel(events):
    ev4 = (events.transpose(2, 0, 1)
           .reshape(5, B, N // 128, 128)
           .transpose(0, 2, 1, 3))
    c0 = _scatter0(ev4)
    c1 = _scatter1(ev4)
    o0 = _post(c0.reshape(B // 2, S, HP, WP))
    o1 = _post(c1.reshape(B // 2, S, HP, WP))
    return jnp.concatenate([o0, o1], axis=0)


# speedup vs baseline: 1.0631x; 1.0631x over previous
import functools

import jax
import jax.numpy as jnp
from jax import lax
from jax.experimental import pallas as pl
from jax.experimental.pallas import tpu as pltpu
from jax.experimental.pallas import tpu_sc as plsc

H, W = 180, 240
S = 32
START_IDX = 2
B = 4
N = 1048576
SEG = N // S
HW = H * W
HV, WV = H // 2, W // 2

HP, WP = 184, 256
HVP, WVP = HP // 2, WP // 2
NC, NS, LANES = 2, 16, 16
NW = NC * NS
PAIRS = B * S
PPW = PAIRS // 2 // NW
CHUNK = 8192
NCHUNK = SEG // CHUNK


CH = 16384
NCH = SEG // CH
NBLK = CH // 128
PLANE = 4 * 8192


def _scatter_body(g, ev_hbm, craw_hbm, bx0, bx1, by0, by1, hist,
                  sx0, sx1, sy0, sy1):
    wid = lax.axis_index("s") * NC + lax.axis_index("c")
    zeros16 = jnp.zeros((LANES,), jnp.int32)
    ones16 = jnp.ones((LANES,), jnp.int32)
    bxs, bys = (bx0, bx1), (by0, by1)
    sxs, syss = (sx0, sx1), (sy0, sy1)
    p0 = wid * PPW
    nslot = PPW * NCH

    def start(t):
        p = g * (PAIRS // 2) + p0 + t // NCH
        b = p // S
        k0 = (p % S) * (SEG // 128) + (t % NCH) * NBLK
        slot = t % 2
        return (pltpu.async_copy(ev_hbm.at[0, pl.ds(k0, NBLK), b],
                                 bxs[slot], sxs[slot]),
                pltpu.async_copy(ev_hbm.at[1, pl.ds(k0, NBLK), b],
                                 bys[slot], syss[slot]))

    @plsc.parallel_loop(0, HP, unroll=2)
    def zero0(r):
        for c in range(WP // LANES):
            hist[r, pl.ds(c * LANES, LANES)] = zeros16

    pend = [start(0)]
    for t in range(nslot):
        if t + 1 < nslot:
            pend.append(start(t + 1))
        cx, cy = pend[t]
        cx.wait()
        cy.wait()
        bx, by = bxs[t % 2], bys[t % 2]

        @plsc.parallel_loop(0, NBLK, unroll=1)
        def scatter_blk(r):
            for c8 in range(8):
                xv = bx[r, pl.ds(c8 * LANES, LANES)].astype(jnp.int32)
                yv = by[r, pl.ds(c8 * LANES, LANES)].astype(jnp.int32)
                plsc.addupdate_scatter(hist, [yv, xv], ones16)

        if t % NCH == NCH - 1:
            pltpu.sync_copy(
                hist, craw_hbm.at[pl.ds((p0 + t // NCH) * HP, HP)])


def _make_scatter(g):
    return functools.partial(
        pl.kernel,
        out_type=jax.ShapeDtypeStruct((PAIRS // 2 * HP, WP), jnp.int32),
        mesh=plsc.VectorSubcoreMesh(core_axis_name="c", subcore_axis_name="s"),
        compiler_params=pltpu.CompilerParams(needs_layout_passes=False),
        scratch_types=[
            pltpu.VMEM((NBLK, 128), jnp.float32),
            pltpu.VMEM((NBLK, 128), jnp.float32),
            pltpu.VMEM((NBLK, 128), jnp.float32),
            pltpu.VMEM((NBLK, 128), jnp.float32),
            pltpu.VMEM((HP, WP), jnp.int32),
            pltpu.SemaphoreType.DMA,
            pltpu.SemaphoreType.DMA,
            pltpu.SemaphoreType.DMA,
            pltpu.SemaphoreType.DMA,
        ],
    )(functools.partial(_scatter_body, g))


_scatter0 = _make_scatter(0)
_scatter1 = _make_scatter(1)


def _aligned_calc(a, D):
    n = S * D
    mean = jnp.sum(a) / n
    var = jnp.sum((a - mean) ** 2) / (n - 1)
    clamp_val = mean + 3.0 * jnp.sqrt(var)
    a = jnp.clip(a, 0.0, clamp_val)
    iS = lax.broadcasted_iota(jnp.int32, (S, S), 0)
    jS = lax.broadcasted_iota(jnp.int32, (S, S), 1)
    TS = (jnp.abs(iS - jS) <= 1).astype(jnp.float32)
    iD = lax.broadcasted_iota(jnp.int32, (D, D), 0)
    jD = lax.broadcasted_iota(jnp.int32, (D, D), 1)
    TD = (jnp.abs(iD - jD) <= 1).astype(jnp.float32)
    box = jnp.dot(jnp.dot(TS, a, preferred_element_type=jnp.float32), TD,
                  preferred_element_type=jnp.float32)
    blur = 0.0625 * box + (0.5 - 0.0625) * a
    dcol = lax.broadcasted_iota(jnp.int32, (D, 1), 0).astype(jnp.float32)
    m = jnp.dot(blur, dcol, preferred_element_type=jnp.float32) / float(SEG)
    sel = lax.broadcasted_iota(jnp.int32, (S, 1), 0) == START_IDX
    start = jnp.sum(jnp.where(sel, m, 0.0))
    dist = (D // 2) - start
    return jnp.round(m - start - dist)


def _post_body(craw_ref, out_ref):
    c_all = craw_ref[0]
    ax_cum = jnp.sum(c_all, axis=1).astype(jnp.float32)[:, :W]
    ay_cum = jnp.sum(c_all, axis=2).astype(jnp.float32)[:, :H]
    seg_iota_x = lax.broadcasted_iota(jnp.int32, (S, W), 0)
    seg_iota_y = lax.broadcasted_iota(jnp.int32, (S, H), 0)
    ax_prev = jnp.concatenate([jnp.zeros((1, W), jnp.float32), ax_cum[:-1]], 0)
    ay_prev = jnp.concatenate([jnp.zeros((1, H), jnp.float32), ay_cum[:-1]], 0)
    along_x = jnp.where(seg_iota_x % PPW == 0, ax_cum, ax_cum - ax_prev)
    along_y = jnp.where(seg_iota_y % PPW == 0, ay_cum, ay_cum - ay_prev)
    a_x = _aligned_calc(along_x, W)
    a_y = _aligned_calc(along_y, H)

    xi = lax.broadcasted_iota(jnp.int32, (WP, WP), 0).astype(jnp.float32)
    xo = lax.broadcasted_iota(jnp.int32, (WP, WP), 1).astype(jnp.float32)
    yi = lax.broadcasted_iota(jnp.int32, (HP, HP), 1).astype(jnp.float32)
    yo = lax.broadcasted_iota(jnp.int32, (HP, HP), 0).astype(jnp.float32)
    qy_i = lax.broadcasted_iota(jnp.int32, (HVP, HP), 1)
    qy_o = lax.broadcasted_iota(jnp.int32, (HVP, HP), 0)
    Qy = (qy_i // 2 == qy_o).astype(jnp.float32)
    qx_i = lax.broadcasted_iota(jnp.int32, (WP, WVP), 0)
    qx_o = lax.broadcasted_iota(jnp.int32, (WP, WVP), 1)
    Qx = (qx_i // 2 == qx_o).astype(jnp.float32)
    sel_iota = lax.broadcasted_iota(jnp.int32, (S, 1), 0)

    def shifted(si):
        sel = sel_iota == si
        ax = jnp.sum(jnp.where(sel, a_x, 0.0))
        ay = jnp.sum(jnp.where(sel, a_y, 0.0))
        Mx = (jnp.clip(xi - ax, 0.0, W - 1.0) == xo).astype(jnp.float32)
        MyT = (jnp.clip(yi - ay, 0.0, H - 1.0) == yo).astype(jnp.float32)
        c_cur = craw_ref[0, si]
        c_prev = craw_ref[0, jnp.maximum(si - 1, 0)]
        keep_prev = jnp.where(jnp.int32(si) % PPW == 0, jnp.int32(0), jnp.int32(1))
        cs = (c_cur - keep_prev * c_prev).astype(jnp.float32)
        sh = jnp.dot(MyT, jnp.dot(cs, Mx, preferred_element_type=jnp.float32),
                     preferred_element_type=jnp.float32)
        occ = (jnp.dot(jnp.dot(Qy, sh, preferred_element_type=jnp.float32), Qx,
                       preferred_element_type=jnp.float32) > 0.0).astype(jnp.float32)
        return sh, occ

    cont0, v0 = shifted(START_IDX)

    def cond(carry):
        si, active, _, _ = carry
        return jnp.logical_and(active, si < S)

    def body(carry):
        si, _, v, cont = carry
        sh, occ = shifted(si)
        vn = jnp.maximum(v, occ)
        vn_cnt = jnp.sum(vn)
        new_info = vn_cnt - jnp.sum(v)
        active = (new_info / vn_cnt) >= 0.01
        cont = jnp.where(active, cont + sh, cont)
        v = jnp.where(active, vn, v)
        return si + 1, active, v, cont

    _, _, _, cont = lax.while_loop(
        cond, body, (jnp.int32(START_IDX + 1), jnp.bool_(True), v0, cont0))
    out_ref[0, 0] = cont[:H, :W]


_post = pl.pallas_call(
    _post_body,
    grid=(B // 2,),
    in_specs=[pl.BlockSpec((1, S, HP, WP), lambda i: (i, 0, 0, 0))],
    out_specs=pl.BlockSpec((1, 1, H, W), lambda i: (i, 0, 0, 0)),
    out_shape=jax.ShapeDtypeStruct((B // 2, 1, H, W), jnp.float32),
)


def kernel(events):
    ev4 = (events.transpose(2, 0, 1)
           .reshape(5, B, N // 128, 128)
           .transpose(0, 2, 1, 3))
    c0 = _scatter0(ev4)
    c1 = _scatter1(ev4)
    o0 = _post(c0.reshape(B // 2, S, HP, WP))
    o1 = _post(c1.reshape(B // 2, S, HP, WP))
    return jnp.concatenate([o0, o1], axis=0)
